# Initial kernel scaffold; baseline (speedup 1.0000x reference)
import jax
import jax.numpy as jnp
from jax.experimental import pallas as pl


def kernel(x, edge_index, W1_l, b1, W1_r, W2_l, b2, W2_r):
    raise NotImplementedError("write your pallas kernel here")


# speedup vs baseline: 5.5210x; 5.5210x over previous
import functools

import jax
import jax.numpy as jnp
from jax import lax
from jax.experimental import pallas as pl
from jax.experimental.pallas import tpu as pltpu
from jax.experimental.pallas import tpu_sc as plsc

_NC = 2
_NS = 16
_NW = _NC * _NS
_CH = 80


@functools.lru_cache(maxsize=None)
def _sc_agg(n_nodes: int, n_edges: int, feat: int, n_pad: int, with_deg: bool):
    epw = n_edges // _NW
    rpt = n_pad // _NS
    assert epw * _NW == n_edges and epw % _CH == 0
    assert rpt * _NS == n_pad and rpt % 8 == 0

    mesh = plsc.VectorSubcoreMesh(core_axis_name="c", subcore_axis_name="s")
    out_type = [jax.ShapeDtypeStruct((_NC, n_pad, feat), jnp.float32)]
    if with_deg:
        out_type.append(jax.ShapeDtypeStruct((_NC, n_pad), jnp.float32))

    scratch_types = [
        pltpu.VMEM((_CH,), jnp.int32),
        pltpu.VMEM((_CH,), jnp.int32),
        pltpu.VMEM((_CH, feat), jnp.float32),
        pltpu.VMEM((_CH,), jnp.float32),
        pltpu.SemaphoreType.DMA,
        pltpu.VMEM_SHARED((n_pad, feat), jnp.float32),
        pltpu.VMEM_SHARED((n_pad,), jnp.float32),
    ]

    @functools.partial(pl.kernel, mesh=mesh, out_type=out_type,
                       scratch_types=scratch_types)
    def agg(x_hbm, src_hbm, dst_hbm, z2_hbm, z1_hbm, *refs):
        if with_deg:
            out_hbm, deg_hbm = refs[0], refs[1]
            refs = refs[2:]
        else:
            out_hbm, deg_hbm = refs[0], None
            refs = refs[1:]
        src_i, dst_i, rows, ones, sem, acc, dacc = refs

        c = lax.axis_index("c")
        s = lax.axis_index("s")
        wid = c * _NS + s

        pltpu.sync_copy(z2_hbm, acc.at[pl.ds(s * rpt, rpt)])
        if with_deg:
            pltpu.sync_copy(z1_hbm, dacc.at[pl.ds(s * rpt, rpt)])

            @pl.loop(0, _CH, step=16)
            def _fill(i):
                ones[pl.ds(i, 16)] = jnp.ones((16,), jnp.float32)

        plsc.subcore_barrier()

        wstart = wid * epw

        @pl.loop(0, epw, step=_CH)
        def _edges(e0):
            base = wstart + e0
            pltpu.sync_copy(src_hbm.at[pl.ds(base, _CH)], src_i)
            pltpu.sync_copy(dst_hbm.at[pl.ds(base, _CH)], dst_i)
            pltpu.async_copy(x_hbm.at[src_i], rows, sem).wait()
            pltpu.sync_copy(rows, acc.at[dst_i], add=True)
            if with_deg:
                pltpu.sync_copy(ones, dacc.at[dst_i], add=True)

        plsc.subcore_barrier()

        pltpu.sync_copy(acc.at[pl.ds(s * rpt, rpt)],
                        out_hbm.at[c, pl.ds(s * rpt, rpt)])
        if with_deg:
            pltpu.sync_copy(dacc.at[pl.ds(s * rpt, rpt)],
                            deg_hbm.at[c, pl.ds(s * rpt, rpt)])

    return agg


@functools.lru_cache(maxsize=None)
def _tc_layer(n_nodes: int, n_pad: int, feat: int, out_feat: int, act: str):
    rblk = 1000
    assert n_nodes % rblk == 0

    def body(p_ref, dg_ref, x_ref, wl_ref, b_ref, wr_ref, o_ref):
        agg = p_ref[0] + p_ref[1]
        deg = dg_ref[0] + dg_ref[1]
        dinv = 1.0 / jnp.maximum(deg, 1.0)
        mean = agg * dinv
        y = jnp.dot(mean, wl_ref[...], preferred_element_type=jnp.float32)
        y = y + jnp.dot(x_ref[...], wr_ref[...],
                        preferred_element_type=jnp.float32)
        y = y + b_ref[...]
        if act == "relu":
            o_ref[...] = jnp.maximum(y, 0.0)
        else:
            m = jnp.max(y, axis=1, keepdims=True)
            lse = jnp.log(jnp.sum(jnp.exp(y - m), axis=1, keepdims=True)) + m
            o_ref[...] = y - lse

    return pl.pallas_call(
        body,
        grid=(n_nodes // rblk,),
        in_specs=[
            pl.BlockSpec((_NC, rblk, feat), lambda i: (0, i, 0)),
            pl.BlockSpec((_NC, rblk, 1), lambda i: (0, i, 0)),
            pl.BlockSpec((rblk, feat), lambda i: (i, 0)),
            pl.BlockSpec((feat, out_feat), lambda i: (0, 0)),
            pl.BlockSpec((1, out_feat), lambda i: (0, 0)),
            pl.BlockSpec((feat, out_feat), lambda i: (0, 0)),
        ],
        out_specs=pl.BlockSpec((rblk, out_feat), lambda i: (i, 0)),
        out_shape=jax.ShapeDtypeStruct((n_nodes, out_feat), jnp.float32),
    )


def kernel(x, edge_index, W1_l, b1, W1_r, W2_l, b2, W2_r):
    n, d = x.shape
    e = edge_index.shape[1]
    h = W1_l.shape[1]
    o = W2_l.shape[1]
    n_pad = 10240

    rpt = n_pad // _NS
    z2 = jnp.zeros((rpt, d), jnp.float32)
    z1 = jnp.zeros((rpt,), jnp.float32)

    src = edge_index[0]
    dst = edge_index[1]
    p1, dg = _sc_agg(n, e, d, n_pad, True)(x, src, dst, z2, z1)
    dg3 = dg.reshape(_NC, n_pad, 1)
    hid = _tc_layer(n, n_pad, d, h, "relu")(
        p1, dg3, x, W1_l, b1.reshape(1, h), W1_r)
    (p2,) = _sc_agg(n, e, h, n_pad, False)(hid, src, dst, z2, z1)
    out = _tc_layer(n, n_pad, h, o, "ls")(
        p2, dg3, hid, W2_l, b2.reshape(1, o), W2_r)
    return out


# speedup vs baseline: 12.2184x; 2.2131x over previous
import functools

import jax
import jax.numpy as jnp
from jax import lax
from jax.experimental import pallas as pl
from jax.experimental.pallas import tpu as pltpu
from jax.experimental.pallas import tpu_sc as plsc

_NC = 2
_NS = 16
_NW = _NC * _NS
_CH = 80


@functools.lru_cache(maxsize=None)
def _sc_agg(n_nodes: int, n_edges: int, feat: int, n_pad: int, with_deg: bool):
    epw = n_edges // _NW
    rpt = n_pad // _NS
    nch = epw // _CH
    assert epw * _NW == n_edges and nch * _CH == epw
    assert rpt * _NS == n_pad and rpt % 8 == 0

    mesh = plsc.VectorSubcoreMesh(core_axis_name="c", subcore_axis_name="s")
    out_type = [jax.ShapeDtypeStruct((_NC, n_pad, feat), jnp.float32)]
    if with_deg:
        out_type.append(jax.ShapeDtypeStruct((_NC, n_pad), jnp.float32))

    scratch_types = [
        pltpu.VMEM((epw,), jnp.int32),
        pltpu.VMEM((nch, _CH), jnp.int32),
        pltpu.VMEM((_CH, feat), jnp.float32),
        pltpu.VMEM((_CH, feat), jnp.float32),
        pltpu.VMEM((_CH,), jnp.float32),
        pltpu.SemaphoreType.DMA,
        pltpu.SemaphoreType.DMA,
        pltpu.VMEM_SHARED((n_pad, feat), jnp.float32),
        pltpu.VMEM_SHARED((n_pad,), jnp.float32),
    ]

    @functools.partial(pl.kernel, mesh=mesh, out_type=out_type,
                       scratch_types=scratch_types)
    def agg(x_hbm, src_hbm, dst3_hbm, z2_hbm, z1_hbm, *refs):
        if with_deg:
            out_hbm, deg_hbm = refs[0], refs[1]
            refs = refs[2:]
        else:
            out_hbm, deg_hbm = refs[0], None
            refs = refs[1:]
        srcbuf, dstbuf, rows0, rows1, ones, sem0, sem1, acc, dacc = refs

        c = lax.axis_index("c")
        s = lax.axis_index("s")
        wid = c * _NS + s

        pltpu.sync_copy(z2_hbm, acc.at[pl.ds(s * rpt, rpt)])
        pltpu.sync_copy(src_hbm.at[pl.ds(wid * epw, epw)], srcbuf)
        pltpu.sync_copy(dst3_hbm.at[wid], dstbuf)
        if with_deg:
            pltpu.sync_copy(z1_hbm, dacc.at[pl.ds(s * rpt, rpt)])

            @pl.loop(0, _CH, step=16)
            def _fill(i):
                ones[pl.ds(i, 16)] = jnp.ones((16,), jnp.float32)

        plsc.subcore_barrier()

        def gather(ci, buf, sem):
            pltpu.async_copy(x_hbm.at[srcbuf.at[pl.ds(ci * _CH, _CH)]],
                             buf, sem)

        def wait(buf, sem):
            pltpu.make_async_copy(x_hbm.at[pl.ds(0, _CH)], buf, sem).wait()

        def scat(ci, buf):
            pltpu.sync_copy(buf, acc.at[dstbuf.at[ci]], add=True)
            if with_deg:
                pltpu.sync_copy(ones, dacc.at[dstbuf.at[ci]], add=True)

        gather(0, rows0, sem0)
        if nch > 1:
            gather(1, rows1, sem1)

        @pl.loop(0, nch, step=2)
        def _pairs(ci):
            wait(rows0, sem0)
            scat(ci, rows0)

            @pl.when(ci + 2 < nch)
            def _():
                gather(ci + 2, rows0, sem0)

            @pl.when(ci + 1 < nch)
            def _():
                wait(rows1, sem1)
                scat(ci + 1, rows1)

                @pl.when(ci + 3 < nch)
                def _():
                    gather(ci + 3, rows1, sem1)

        plsc.subcore_barrier()

        pltpu.sync_copy(acc.at[pl.ds(s * rpt, rpt)],
                        out_hbm.at[c, pl.ds(s * rpt, rpt)])
        if with_deg:
            pltpu.sync_copy(dacc.at[pl.ds(s * rpt, rpt)],
                            deg_hbm.at[c, pl.ds(s * rpt, rpt)])

    return agg


@functools.lru_cache(maxsize=None)
def _tc_layer(n_nodes: int, n_pad: int, feat: int, out_feat: int, act: str):
    rblk = 1000
    assert n_nodes % rblk == 0

    def body(p_ref, dg_ref, x_ref, wl_ref, b_ref, wr_ref, o_ref):
        agg = p_ref[0] + p_ref[1]
        deg = dg_ref[0] + dg_ref[1]
        dinv = 1.0 / jnp.maximum(deg, 1.0)
        mean = agg * dinv
        y = jnp.dot(mean, wl_ref[...], preferred_element_type=jnp.float32)
        y = y + jnp.dot(x_ref[...], wr_ref[...],
                        preferred_element_type=jnp.float32)
        y = y + b_ref[...]
        if act == "relu":
            o_ref[...] = jnp.maximum(y, 0.0)
        else:
            m = jnp.max(y, axis=1, keepdims=True)
            lse = jnp.log(jnp.sum(jnp.exp(y - m), axis=1, keepdims=True)) + m
            o_ref[...] = y - lse

    return pl.pallas_call(
        body,
        grid=(n_nodes // rblk,),
        in_specs=[
            pl.BlockSpec((_NC, rblk, feat), lambda i: (0, i, 0)),
            pl.BlockSpec((_NC, rblk, 1), lambda i: (0, i, 0)),
            pl.BlockSpec((rblk, feat), lambda i: (i, 0)),
            pl.BlockSpec((feat, out_feat), lambda i: (0, 0)),
            pl.BlockSpec((1, out_feat), lambda i: (0, 0)),
            pl.BlockSpec((feat, out_feat), lambda i: (0, 0)),
        ],
        out_specs=pl.BlockSpec((rblk, out_feat), lambda i: (i, 0)),
        out_shape=jax.ShapeDtypeStruct((n_nodes, out_feat), jnp.float32),
    )


def kernel(x, edge_index, W1_l, b1, W1_r, W2_l, b2, W2_r):
    n, d = x.shape
    e = edge_index.shape[1]
    h = W1_l.shape[1]
    o = W2_l.shape[1]
    n_pad = 10240

    rpt = n_pad // _NS
    z2 = jnp.zeros((rpt, d), jnp.float32)
    z1 = jnp.zeros((rpt,), jnp.float32)

    src = edge_index[0]
    dst3 = edge_index[1].reshape(_NW, (e // _NW) // _CH, _CH)
    p1, dg = _sc_agg(n, e, d, n_pad, True)(x, src, dst3, z2, z1)
    dg3 = dg.reshape(_NC, n_pad, 1)
    hid = _tc_layer(n, n_pad, d, h, "relu")(
        p1, dg3, x, W1_l, b1.reshape(1, h), W1_r)
    (p2,) = _sc_agg(n, e, h, n_pad, False)(hid, src, dst3, z2, z1)
    out = _tc_layer(n, n_pad, h, o, "ls")(
        p2, dg3, hid, W2_l, b2.reshape(1, o), W2_r)
    return out
